# baseline (speedup 1.0000x reference)
import functools

import jax
import jax.numpy as jnp
from jax import lax
from jax.experimental import pallas as pl
from jax.experimental.pallas import tpu as pltpu
from jax.experimental.pallas import tpu_sc as plsc

N = 8192
D = 256
H = 2
C = 256

QB = 256
KB = 512

RPT = 512
RPC = 32
NCH = RPT // RPC
FBLK = 32
NF = C // FBLK


def _top3_body(q_ref, k_ref, idx_ref):
    q = q_ref[...]
    qsq = jnp.sum(q * q, axis=1)

    def step(t, carry):
        v1, v2, v3, i1, i2, i3 = carry
        k = k_ref[pl.ds(t * KB, KB), :]
        ksq = jnp.sum(k * k, axis=1)
        dot = jax.lax.dot_general(
            q, k, (((1,), (1,)), ((), ())),
            preferred_element_type=jnp.float32)
        d2 = qsq[:, None] + ksq[None, :] - 2.0 * dot
        dist = jnp.sqrt(jnp.maximum(d2, 0.0))
        col = jax.lax.broadcasted_iota(jnp.int32, (QB, KB), 1) + t * KB

        def block_min(dmat):
            m = jnp.min(dmat, axis=1)
            idx = jnp.min(jnp.where(dmat == m[:, None], col, N), axis=1)
            dmat2 = jnp.where(col == idx[:, None], jnp.inf, dmat)
            return m, idx, dmat2

        m1, j1, dist = block_min(dist)
        m2, j2, dist = block_min(dist)
        m3, j3, dist = block_min(dist)

        def insert(m, j, v1, v2, v3, i1, i2, i3):
            c1 = m < v1
            c2 = m < v2
            c3 = m < v3
            nv3 = jnp.where(c3, jnp.where(c2, v2, m), v3)
            ni3 = jnp.where(c3, jnp.where(c2, i2, j), i3)
            nv2 = jnp.where(c2, jnp.where(c1, v1, m), v2)
            ni2 = jnp.where(c2, jnp.where(c1, i1, j), i2)
            nv1 = jnp.where(c1, m, v1)
            ni1 = jnp.where(c1, j, i1)
            return nv1, nv2, nv3, ni1, ni2, ni3

        v1, v2, v3, i1, i2, i3 = insert(m1, j1, v1, v2, v3, i1, i2, i3)
        v1, v2, v3, i1, i2, i3 = insert(m2, j2, v1, v2, v3, i1, i2, i3)
        v1, v2, v3, i1, i2, i3 = insert(m3, j3, v1, v2, v3, i1, i2, i3)
        return v1, v2, v3, i1, i2, i3

    inf = jnp.full((QB,), jnp.inf, dtype=jnp.float32)
    zero = jnp.zeros((QB,), dtype=jnp.int32)
    v1, v2, v3, i1, i2, i3 = jax.lax.fori_loop(
        0, N // KB, step, (inf, inf, inf, zero, zero, zero))
    idx_ref[...] = jnp.stack([i1, i2, i3, i1, i1, i1, i1, i1], axis=0)


def _top3(prototypes):
    return pl.pallas_call(
        _top3_body,
        grid=(N // QB,),
        in_specs=[
            pl.BlockSpec((QB, D), lambda i: (i, 0)),
            pl.BlockSpec((N, D), lambda i: (0, 0)),
        ],
        out_specs=pl.BlockSpec((8, QB), lambda i: (0, i)),
        out_shape=jax.ShapeDtypeStruct((8, N), jnp.int32),
    )(prototypes, prototypes)


def _leaky(x):
    return jnp.where(x >= 0, x, 0.2 * x)


def _pre_body(p_ref, w_ref, asrcw_ref, adstw_ref,
              xw_ref, accinit_ref, deninit_ref, asrcv_ref, adstv_ref):
    p = p_ref[...]
    w = w_ref[...]
    xw = jnp.dot(p, w, preferred_element_type=jnp.float32)
    xw_ref[...] = jnp.stack(
        [xw[:, g * FBLK:(g + 1) * FBLK] for g in range(H * NF)], axis=0)
    asrcw = asrcw_ref[...]
    adstw = adstw_ref[...]
    h0 = xw[:, :C]
    h1 = xw[:, C:]
    as0 = jnp.sum(h0 * asrcw[0][None, :], axis=1)
    as1 = jnp.sum(h1 * asrcw[1][None, :], axis=1)
    ad0 = jnp.sum(h0 * adstw[0][None, :], axis=1)
    ad1 = jnp.sum(h1 * adstw[1][None, :], axis=1)
    es0 = jnp.exp(_leaky(as0 + ad0))
    es1 = jnp.exp(_leaky(as1 + ad1))
    accinit = jnp.concatenate([h0 * es0[:, None], h1 * es1[:, None]], axis=1)
    accinit_ref[...] = jnp.stack(
        [accinit[:, g * FBLK:(g + 1) * FBLK] for g in range(H * NF)], axis=0)
    col0 = jnp.stack([es0, es1], axis=0)[:, :, None]
    deninit_ref[...] = jnp.concatenate(
        [col0, jnp.zeros((H, QB, 15), jnp.float32)], axis=2)
    asrcv_ref[...] = jnp.stack([as0, as1], axis=0)[:, None, :]
    adstv_ref[...] = jnp.stack([ad0, ad1], axis=0)[:, None, :]


def _pre(prototypes, W, att_src, att_dst):
    return pl.pallas_call(
        _pre_body,
        grid=(N // QB,),
        in_specs=[
            pl.BlockSpec((QB, D), lambda i: (i, 0)),
            pl.BlockSpec((D, H * C), lambda i: (0, 0)),
            pl.BlockSpec((H, C), lambda i: (0, 0)),
            pl.BlockSpec((H, C), lambda i: (0, 0)),
        ],
        out_specs=[
            pl.BlockSpec((H * NF, QB, FBLK), lambda i: (0, i, 0)),
            pl.BlockSpec((H * NF, QB, FBLK), lambda i: (0, i, 0)),
            pl.BlockSpec((H, QB, 16), lambda i: (0, i, 0)),
            pl.BlockSpec((H, 1, QB), lambda i: (0, 0, i)),
            pl.BlockSpec((H, 1, QB), lambda i: (0, 0, i)),
        ],
        out_shape=[
            jax.ShapeDtypeStruct((H * NF, N, FBLK), jnp.float32),
            jax.ShapeDtypeStruct((H * NF, N, FBLK), jnp.float32),
            jax.ShapeDtypeStruct((H, N, 16), jnp.float32),
            jax.ShapeDtypeStruct((H, 1, N), jnp.float32),
            jax.ShapeDtypeStruct((H, 1, N), jnp.float32),
        ],
    )(prototypes, W, att_src, att_dst)


def _sc_body(xw_hbm, accinit_hbm, deninit_hbm, asrc_hbm, adst_hbm,
             d0_hbm, d1_hbm, d2_hbm,
             acc_out, den_out,
             acc_s, den_s,
             xw_v, msgs_v, denrows_v, ex0_v, ex1_v, ex2_v,
             db0_v, db1_v, db2_v, asrc_v, adst_v, idxc_v):
    exs_v = (ex0_v, ex1_v, ex2_v)
    dbs_v = (db0_v, db1_v, db2_v)
    c = lax.axis_index("c")
    s = lax.axis_index("s")
    row0 = s * RPT
    lanes = lax.iota(jnp.int32, 16)
    zeros16 = jnp.zeros((16,), jnp.float32)
    izeros16 = jnp.zeros((16,), jnp.int32)

    pltpu.sync_copy(adst_hbm.at[c, 0], adst_v)
    pltpu.sync_copy(asrc_hbm.at[c, 0, pl.ds(row0, RPT)], asrc_v)
    for k, dbuf in enumerate((d0_hbm, d1_hbm, d2_hbm)):
        pltpu.sync_copy(dbuf.at[pl.ds(row0, RPT)], dbs_v[k])

    for k in range(3):
        for i in range(RPT // 16):
            dk = dbs_v[k][pl.ds(i * 16, 16)]
            adst_g = plsc.load_gather(adst_v, [dk])
            alpha = asrc_v[pl.ds(i * 16, 16)] + adst_g
            ex = jnp.exp(jnp.where(alpha >= 0, alpha, 0.2 * alpha))
            rowid = row0 + i * 16 + lanes
            exs_v[k][pl.ds(i * 16, 16)] = jnp.where(dk == rowid, 0.0, ex)

    for e in range(3 * RPC):
        denrows_v[e, :] = zeros16

    for f in range(NF):
        g = c * NF + f
        pltpu.sync_copy(accinit_hbm.at[g, pl.ds(row0, RPT), :],
                        acc_s.at[pl.ds(row0, RPT), :])
        if f == 0:
            pltpu.sync_copy(deninit_hbm.at[c, pl.ds(row0, RPT), :],
                            den_s.at[pl.ds(row0, RPT), :])
        pltpu.sync_copy(xw_hbm.at[g, pl.ds(row0, RPT), :], xw_v)
        plsc.subcore_barrier()

        def chunk(j, _):
            jr = j * RPC
            for k in range(3):
                for i in range(RPC // 16):
                    idxc_v[pl.ds(k * RPC + i * 16, 16)] = \
                        dbs_v[k][pl.ds(jr + i * 16, 16)]
            for k in range(3):
                for r in range(RPC):
                    e = k * RPC + r
                    exb = plsc.load_gather(
                        exs_v[k], [jnp.full((16,), jr + r, jnp.int32)])
                    for b in range(FBLK // 16):
                        msgs_v[e, pl.ds(b * 16, 16)] = \
                            xw_v[jr + r, pl.ds(b * 16, 16)] * exb
            if f == 0:
                for k in range(3):
                    for i in range(RPC // 16):
                        e0 = k * RPC + i * 16
                        evec = e0 + lanes
                        exv = exs_v[k][pl.ds(jr + i * 16, 16)]
                        plsc.store_scatter(denrows_v, [evec, izeros16], exv)
                pltpu.sync_copy(denrows_v, den_s.at[idxc_v], add=True)
            pltpu.sync_copy(msgs_v, acc_s.at[idxc_v], add=True)
            return 0

        lax.fori_loop(0, NCH, chunk, 0)
        plsc.subcore_barrier()
        pltpu.sync_copy(acc_s.at[pl.ds(row0, RPT), :],
                        acc_out.at[c, f, pl.ds(row0, RPT), :])
        if f == 0:
            pltpu.sync_copy(den_s.at[pl.ds(row0, RPT), :],
                            den_out.at[c, pl.ds(row0, RPT), :])
        plsc.subcore_barrier()


@functools.partial(
    pl.kernel,
    out_type=[
        jax.ShapeDtypeStruct((H, NF, N, FBLK), jnp.float32),
        jax.ShapeDtypeStruct((H, N, 16), jnp.float32),
    ],
    mesh=plsc.VectorSubcoreMesh(core_axis_name="c", subcore_axis_name="s"),
    compiler_params=pltpu.CompilerParams(needs_layout_passes=False),
    scratch_types=[
        pltpu.VMEM_SHARED((N, FBLK), jnp.float32),
        pltpu.VMEM_SHARED((N, 16), jnp.float32),
        pltpu.VMEM((RPT, FBLK), jnp.float32),
        pltpu.VMEM((3 * RPC, FBLK), jnp.float32),
        pltpu.VMEM((3 * RPC, 16), jnp.float32),
        pltpu.VMEM((RPT,), jnp.float32),
        pltpu.VMEM((RPT,), jnp.float32),
        pltpu.VMEM((RPT,), jnp.float32),
        pltpu.VMEM((RPT,), jnp.int32),
        pltpu.VMEM((RPT,), jnp.int32),
        pltpu.VMEM((RPT,), jnp.int32),
        pltpu.VMEM((RPT,), jnp.float32),
        pltpu.VMEM((N,), jnp.float32),
        pltpu.VMEM((3 * RPC,), jnp.int32),
    ],
)
def _sc_scatter(xw_hbm, accinit_hbm, deninit_hbm, asrc_hbm, adst_hbm,
                d0_hbm, d1_hbm, d2_hbm, acc_out, den_out, *scratch):
    _sc_body(xw_hbm, accinit_hbm, deninit_hbm, asrc_hbm, adst_hbm,
             d0_hbm, d1_hbm, d2_hbm, acc_out, den_out, *scratch)


def _post_body(acc_ref, den_ref, p_ref, bias_ref, gamma_ref, beta_ref,
               out_ref):
    acc = acc_ref[...]
    den = den_ref[...]
    d0 = jnp.sum(den[0], axis=1)
    d1 = jnp.sum(den[1], axis=1)
    a0 = jnp.concatenate([acc[0, f] for f in range(NF)], axis=1)
    a1 = jnp.concatenate([acc[1, f] for f in range(NF)], axis=1)
    out = 0.5 * (a0 / d0[:, None] + a1 / d1[:, None]) + bias_ref[...]
    mu = jnp.mean(out, axis=1, keepdims=True)
    var = jnp.mean((out - mu) ** 2, axis=1, keepdims=True)
    out = (out - mu) / jnp.sqrt(var + 1e-5) * gamma_ref[...] + beta_ref[...]
    out_ref[...] = p_ref[...] + jnp.maximum(out, 0.0)


def _post(acc, den, prototypes, bias, gamma, beta):
    return pl.pallas_call(
        _post_body,
        grid=(N // QB,),
        in_specs=[
            pl.BlockSpec((H, NF, QB, FBLK), lambda i: (0, 0, i, 0)),
            pl.BlockSpec((H, QB, 16), lambda i: (0, i, 0)),
            pl.BlockSpec((QB, C), lambda i: (i, 0)),
            pl.BlockSpec((1, C), lambda i: (0, 0)),
            pl.BlockSpec((1, C), lambda i: (0, 0)),
            pl.BlockSpec((1, C), lambda i: (0, 0)),
        ],
        out_specs=pl.BlockSpec((QB, C), lambda i: (i, 0)),
        out_shape=jax.ShapeDtypeStruct((N, C), jnp.float32),
    )(acc, den, prototypes, bias.reshape(1, C), gamma.reshape(1, C),
      beta.reshape(1, C))


def kernel(prototypes, labels, W, att_src, att_dst, bias, gamma, beta):
    idx8 = _top3(prototypes)
    xw, accinit, deninit, asrcv, adstv = _pre(
        prototypes, W, att_src, att_dst)
    acc, den = _sc_scatter(xw, accinit, deninit, asrcv, adstv,
                           idx8[0], idx8[1], idx8[2])
    return _post(acc, den, prototypes, bias, gamma, beta)


# speedup vs baseline: 1.0131x; 1.0131x over previous
import functools

import jax
import jax.numpy as jnp
from jax import lax
from jax.experimental import pallas as pl
from jax.experimental.pallas import tpu as pltpu
from jax.experimental.pallas import tpu_sc as plsc

N = 8192
D = 256
H = 2
C = 256

QB = 256
KB = 512

RPT = 512
RPC = 32
NCH = RPT // RPC
FBLK = 32
DSH = 5
DROWS = N // FBLK
NF = C // FBLK


def _top3_body(q_ref, k_ref, idx_ref):
    q = q_ref[...]
    qsq = jnp.sum(q * q, axis=1)

    def step(t, carry):
        v1, v2, v3, i1, i2, i3 = carry
        k = k_ref[pl.ds(t * KB, KB), :]
        ksq = jnp.sum(k * k, axis=1)
        dot = jax.lax.dot_general(
            q, k, (((1,), (1,)), ((), ())),
            preferred_element_type=jnp.float32)
        d2 = qsq[:, None] + ksq[None, :] - 2.0 * dot
        dist = jnp.sqrt(jnp.maximum(d2, 0.0))
        col = jax.lax.broadcasted_iota(jnp.int32, (QB, KB), 1) + t * KB

        def block_min(dmat):
            m = jnp.min(dmat, axis=1)
            idx = jnp.min(jnp.where(dmat == m[:, None], col, N), axis=1)
            dmat2 = jnp.where(col == idx[:, None], jnp.inf, dmat)
            return m, idx, dmat2

        m1, j1, dist = block_min(dist)
        m2, j2, dist = block_min(dist)
        m3, j3, dist = block_min(dist)

        def insert(m, j, v1, v2, v3, i1, i2, i3):
            c1 = m < v1
            c2 = m < v2
            c3 = m < v3
            nv3 = jnp.where(c3, jnp.where(c2, v2, m), v3)
            ni3 = jnp.where(c3, jnp.where(c2, i2, j), i3)
            nv2 = jnp.where(c2, jnp.where(c1, v1, m), v2)
            ni2 = jnp.where(c2, jnp.where(c1, i1, j), i2)
            nv1 = jnp.where(c1, m, v1)
            ni1 = jnp.where(c1, j, i1)
            return nv1, nv2, nv3, ni1, ni2, ni3

        v1, v2, v3, i1, i2, i3 = insert(m1, j1, v1, v2, v3, i1, i2, i3)
        v1, v2, v3, i1, i2, i3 = insert(m2, j2, v1, v2, v3, i1, i2, i3)
        v1, v2, v3, i1, i2, i3 = insert(m3, j3, v1, v2, v3, i1, i2, i3)
        return v1, v2, v3, i1, i2, i3

    inf = jnp.full((QB,), jnp.inf, dtype=jnp.float32)
    zero = jnp.zeros((QB,), dtype=jnp.int32)
    v1, v2, v3, i1, i2, i3 = jax.lax.fori_loop(
        0, N // KB, step, (inf, inf, inf, zero, zero, zero))
    idx_ref[...] = jnp.stack([i1, i2, i3, i1, i1, i1, i1, i1], axis=0)


def _top3(prototypes):
    return pl.pallas_call(
        _top3_body,
        grid=(N // QB,),
        in_specs=[
            pl.BlockSpec((QB, D), lambda i: (i, 0)),
            pl.BlockSpec((N, D), lambda i: (0, 0)),
        ],
        out_specs=pl.BlockSpec((8, QB), lambda i: (0, i)),
        out_shape=jax.ShapeDtypeStruct((8, N), jnp.int32),
    )(prototypes, prototypes)


def _leaky(x):
    return jnp.where(x >= 0, x, 0.2 * x)


def _pre_body(p_ref, w_ref, asrcw_ref, adstw_ref,
              xw_ref, accinit_ref, deninit_ref, asrcv_ref, adstv_ref):
    p = p_ref[...]
    w = w_ref[...]
    xw = jnp.dot(p, w, preferred_element_type=jnp.float32)
    xw_ref[...] = jnp.stack(
        [xw[:, g * FBLK:(g + 1) * FBLK] for g in range(H * NF)], axis=0)
    asrcw = asrcw_ref[...]
    adstw = adstw_ref[...]
    h0 = xw[:, :C]
    h1 = xw[:, C:]
    as0 = jnp.sum(h0 * asrcw[0][None, :], axis=1)
    as1 = jnp.sum(h1 * asrcw[1][None, :], axis=1)
    ad0 = jnp.sum(h0 * adstw[0][None, :], axis=1)
    ad1 = jnp.sum(h1 * adstw[1][None, :], axis=1)
    es0 = jnp.exp(_leaky(as0 + ad0))
    es1 = jnp.exp(_leaky(as1 + ad1))
    accinit = jnp.concatenate([h0 * es0[:, None], h1 * es1[:, None]], axis=1)
    accinit_ref[...] = jnp.stack(
        [accinit[:, g * FBLK:(g + 1) * FBLK] for g in range(H * NF)], axis=0)
    col0 = jnp.stack([es0, es1], axis=0)[:, :, None]
    deninit_ref[...] = jnp.concatenate(
        [col0, jnp.zeros((H, QB, 15), jnp.float32)], axis=2)
    asrcv_ref[...] = jnp.stack([as0, as1], axis=0)[:, None, :]
    adstv_ref[...] = jnp.stack([ad0, ad1], axis=0)[:, None, :]


def _pre(prototypes, W, att_src, att_dst):
    return pl.pallas_call(
        _pre_body,
        grid=(N // QB,),
        in_specs=[
            pl.BlockSpec((QB, D), lambda i: (i, 0)),
            pl.BlockSpec((D, H * C), lambda i: (0, 0)),
            pl.BlockSpec((H, C), lambda i: (0, 0)),
            pl.BlockSpec((H, C), lambda i: (0, 0)),
        ],
        out_specs=[
            pl.BlockSpec((H * NF, QB, FBLK), lambda i: (0, i, 0)),
            pl.BlockSpec((H * NF, QB, FBLK), lambda i: (0, i, 0)),
            pl.BlockSpec((H, QB, 16), lambda i: (0, i, 0)),
            pl.BlockSpec((H, 1, QB), lambda i: (0, 0, i)),
            pl.BlockSpec((H, 1, QB), lambda i: (0, 0, i)),
        ],
        out_shape=[
            jax.ShapeDtypeStruct((H * NF, N, FBLK), jnp.float32),
            jax.ShapeDtypeStruct((H * NF, N, FBLK), jnp.float32),
            jax.ShapeDtypeStruct((H, N, 16), jnp.float32),
            jax.ShapeDtypeStruct((H, 1, N), jnp.float32),
            jax.ShapeDtypeStruct((H, 1, N), jnp.float32),
        ],
    )(prototypes, W, att_src, att_dst)


def _sc_body(xw_hbm, accinit_hbm, deninit_hbm, asrc_hbm, adst_hbm,
             d0_hbm, d1_hbm, d2_hbm,
             acc_out, den_out,
             acc_s, den_s,
             xw_v, msgs_v, denrows_v, ex0_v, ex1_v, ex2_v,
             db0_v, db1_v, db2_v, asrc_v, adst_v, idxc_v, didxc_v):
    exs_v = (ex0_v, ex1_v, ex2_v)
    dbs_v = (db0_v, db1_v, db2_v)
    c = lax.axis_index("c")
    s = lax.axis_index("s")
    row0 = s * RPT
    lanes = lax.iota(jnp.int32, 16)
    zeros16 = jnp.zeros((16,), jnp.float32)
    izeros16 = jnp.zeros((16,), jnp.int32)

    pltpu.sync_copy(adst_hbm.at[c, 0], adst_v)
    pltpu.sync_copy(asrc_hbm.at[c, 0, pl.ds(row0, RPT)], asrc_v)
    for k, dbuf in enumerate((d0_hbm, d1_hbm, d2_hbm)):
        pltpu.sync_copy(dbuf.at[pl.ds(row0, RPT)], dbs_v[k])

    for k in range(3):
        for i in range(RPT // 16):
            dk = dbs_v[k][pl.ds(i * 16, 16)]
            adst_g = plsc.load_gather(adst_v, [dk])
            alpha = asrc_v[pl.ds(i * 16, 16)] + adst_g
            ex = jnp.exp(jnp.where(alpha >= 0, alpha, 0.2 * alpha))
            rowid = row0 + i * 16 + lanes
            exs_v[k][pl.ds(i * 16, 16)] = jnp.where(dk == rowid, 0.0, ex)

    for e in range(3 * RPC):
        denrows_v[e, :] = zeros16

    for f in range(NF):
        g = c * NF + f
        pltpu.sync_copy(accinit_hbm.at[g, pl.ds(row0, RPT), :],
                        acc_s.at[pl.ds(row0, RPT), :])
        if f == 0:
            pltpu.sync_copy(deninit_hbm.at[c, pl.ds(row0, RPT), :],
                            den_s.at[pl.ds(row0, RPT), :])
        pltpu.sync_copy(xw_hbm.at[g, pl.ds(row0, RPT), :], xw_v)
        plsc.subcore_barrier()

        def chunk(j, _):
            jr = j * RPC
            for k in range(3):
                for i in range(RPC // 16):
                    idxc_v[pl.ds(k * RPC + i * 16, 16)] = \
                        dbs_v[k][pl.ds(jr + i * 16, 16)]
            for r in range(RPC):
                xwb = [xw_v[jr + r, pl.ds(b * 16, 16)]
                       for b in range(FBLK // 16)]
                for k in range(3):
                    e = k * RPC + r
                    exb = plsc.load_gather(
                        exs_v[k], [jnp.full((16,), jr + r, jnp.int32)])
                    for b in range(FBLK // 16):
                        msgs_v[e, pl.ds(b * 16, 16)] = xwb[b] * exb
            if f == 0:
                for k in range(3):
                    for i in range(RPC // 16):
                        e0 = k * RPC + i * 16
                        evec = e0 + lanes
                        exv = exs_v[k][pl.ds(jr + i * 16, 16)]
                        plsc.store_scatter(denrows_v, [evec, izeros16], exv)
                pltpu.sync_copy(denrows_v, den_s.at[idxc_v], add=True)
            pltpu.sync_copy(msgs_v, acc_s.at[idxc_v], add=True)
            return 0

        lax.fori_loop(0, NCH, chunk, 0)
        plsc.subcore_barrier()
        pltpu.sync_copy(acc_s.at[pl.ds(row0, RPT), :],
                        acc_out.at[c, f, pl.ds(row0, RPT), :])
        if f == 0:
            pltpu.sync_copy(den_s.at[pl.ds(row0, RPT), :],
                            den_out.at[c, pl.ds(row0, RPT), :])
        plsc.subcore_barrier()


@functools.lru_cache(maxsize=1)
def _sc_kernel():
    return functools.partial(
        pl.kernel,
        out_type=[
        jax.ShapeDtypeStruct((H, NF, N, FBLK), jnp.float32),
        jax.ShapeDtypeStruct((H, N, 16), jnp.float32),
    ],
        mesh=plsc.VectorSubcoreMesh(core_axis_name="c", subcore_axis_name="s"),
        compiler_params=pltpu.CompilerParams(needs_layout_passes=False),
        scratch_types=[
        pltpu.VMEM_SHARED((N, FBLK), jnp.float32),
            pltpu.VMEM_SHARED((N, 16), jnp.float32),
            pltpu.VMEM((RPT, FBLK), jnp.float32),
            pltpu.VMEM((3 * RPC, FBLK), jnp.float32),
            pltpu.VMEM((3 * RPC, 16), jnp.float32),
            pltpu.VMEM((RPT,), jnp.float32),
            pltpu.VMEM((RPT,), jnp.float32),
            pltpu.VMEM((RPT,), jnp.float32),
            pltpu.VMEM((RPT,), jnp.int32),
            pltpu.VMEM((RPT,), jnp.int32),
            pltpu.VMEM((RPT,), jnp.int32),
            pltpu.VMEM((RPT,), jnp.float32),
            pltpu.VMEM((N,), jnp.float32),
            pltpu.VMEM((3 * RPC,), jnp.int32),
            pltpu.VMEM((3 * RPC,), jnp.int32),
        ],
    )(_sc_body)


def _sc_scatter(*args):
    return _sc_kernel()(*args)


def _post_body(acc_ref, d0_ref, d1_ref, p_ref, bias_ref, gamma_ref, beta_ref,
               out_ref):
    acc = acc_ref[...]
    d0 = d0_ref[...]
    d1 = d1_ref[...]
    a0 = jnp.concatenate([acc[0, f] for f in range(NF)], axis=1)
    a1 = jnp.concatenate([acc[1, f] for f in range(NF)], axis=1)
    out = 0.5 * (a0 / d0 + a1 / d1) + bias_ref[...]
    mu = jnp.mean(out, axis=1, keepdims=True)
    var = jnp.mean((out - mu) ** 2, axis=1, keepdims=True)
    out = (out - mu) / jnp.sqrt(var + 1e-5) * gamma_ref[...] + beta_ref[...]
    out_ref[...] = p_ref[...] + jnp.maximum(out, 0.0)


def _post(acc, den, prototypes, bias, gamma, beta):
    return pl.pallas_call(
        _post_body,
        grid=(N // QB,),
        in_specs=[
            pl.BlockSpec((H, NF, QB, FBLK), lambda i: (0, 0, i, 0)),
            pl.BlockSpec((QB, 1), lambda i: (i, 0)),
            pl.BlockSpec((QB, 1), lambda i: (i, 0)),
            pl.BlockSpec((QB, C), lambda i: (i, 0)),
            pl.BlockSpec((1, C), lambda i: (0, 0)),
            pl.BlockSpec((1, C), lambda i: (0, 0)),
            pl.BlockSpec((1, C), lambda i: (0, 0)),
        ],
        out_specs=pl.BlockSpec((QB, C), lambda i: (i, 0)),
        out_shape=jax.ShapeDtypeStruct((N, C), jnp.float32),
    )(acc, den[0, :, :1], den[1, :, :1], prototypes,
      bias.reshape(1, C), gamma.reshape(1, C), beta.reshape(1, C))


def kernel(prototypes, labels, W, att_src, att_dst, bias, gamma, beta):
    idx8 = _top3(prototypes)
    xw, accinit, deninit, asrcv, adstv = _pre(
        prototypes, W, att_src, att_dst)
    acc, den = _sc_scatter(xw, accinit, deninit, asrcv, adstv,
                           idx8[0], idx8[1], idx8[2])
    return _post(acc, den, prototypes, bias, gamma, beta)


# speedup vs baseline: 1.0427x; 1.0292x over previous
import functools

import jax
import jax.numpy as jnp
from jax import lax
from jax.experimental import pallas as pl
from jax.experimental.pallas import tpu as pltpu
from jax.experimental.pallas import tpu_sc as plsc

N = 8192
D = 256
H = 2
C = 256

QB = 256
KB = 512

RPT = 512
RPC = 32
NCH = RPT // RPC
FBLK = 32
DSH = 5
DROWS = N // FBLK
NF = C // FBLK


def _top3_body(q_ref, k_ref, idx_ref):
    q = q_ref[...]
    qsq = jnp.sum(q * q, axis=1)

    def step(t, carry):
        v1, v2, v3, i1, i2, i3 = carry
        k = k_ref[pl.ds(t * KB, KB), :]
        ksq = jnp.sum(k * k, axis=1)
        dot = jax.lax.dot_general(
            q, k, (((1,), (1,)), ((), ())),
            preferred_element_type=jnp.float32)
        d2 = qsq[:, None] + ksq[None, :] - 2.0 * dot
        dist = jnp.sqrt(jnp.maximum(d2, 0.0))
        col = jax.lax.broadcasted_iota(jnp.int32, (QB, KB), 1) + t * KB

        def block_min(dmat):
            m = jnp.min(dmat, axis=1)
            idx = jnp.min(jnp.where(dmat == m[:, None], col, N), axis=1)
            dmat2 = jnp.where(col == idx[:, None], jnp.inf, dmat)
            return m, idx, dmat2

        m1, j1, dist = block_min(dist)
        m2, j2, dist = block_min(dist)
        m3, j3, dist = block_min(dist)

        def insert(m, j, v1, v2, v3, i1, i2, i3):
            c1 = m < v1
            c2 = m < v2
            c3 = m < v3
            nv3 = jnp.where(c3, jnp.where(c2, v2, m), v3)
            ni3 = jnp.where(c3, jnp.where(c2, i2, j), i3)
            nv2 = jnp.where(c2, jnp.where(c1, v1, m), v2)
            ni2 = jnp.where(c2, jnp.where(c1, i1, j), i2)
            nv1 = jnp.where(c1, m, v1)
            ni1 = jnp.where(c1, j, i1)
            return nv1, nv2, nv3, ni1, ni2, ni3

        v1, v2, v3, i1, i2, i3 = insert(m1, j1, v1, v2, v3, i1, i2, i3)
        v1, v2, v3, i1, i2, i3 = insert(m2, j2, v1, v2, v3, i1, i2, i3)
        v1, v2, v3, i1, i2, i3 = insert(m3, j3, v1, v2, v3, i1, i2, i3)
        return v1, v2, v3, i1, i2, i3

    inf = jnp.full((QB,), jnp.inf, dtype=jnp.float32)
    zero = jnp.zeros((QB,), dtype=jnp.int32)
    v1, v2, v3, i1, i2, i3 = jax.lax.fori_loop(
        0, N // KB, step, (inf, inf, inf, zero, zero, zero))
    idx_ref[...] = jnp.stack([i1, i2, i3, i1, i1, i1, i1, i1], axis=0)


def _leaky(x):
    return jnp.where(x >= 0, x, 0.2 * x)


def _fused_body(p_ref, k_ref, w_ref, asrcw_ref, adstw_ref,
                idx_ref, xw_ref, accinit_ref, deninit_ref,
                asrcv_ref, adstv_ref):
    _top3_body(p_ref, k_ref, idx_ref)
    p = p_ref[...]
    w = w_ref[...]
    xw = jnp.dot(p, w, preferred_element_type=jnp.float32)
    xw_ref[...] = jnp.stack(
        [xw[:, g * FBLK:(g + 1) * FBLK] for g in range(H * NF)], axis=0)
    asrcw = asrcw_ref[...]
    adstw = adstw_ref[...]
    h0 = xw[:, :C]
    h1 = xw[:, C:]
    as0 = jnp.sum(h0 * asrcw[0][None, :], axis=1)
    as1 = jnp.sum(h1 * asrcw[1][None, :], axis=1)
    ad0 = jnp.sum(h0 * adstw[0][None, :], axis=1)
    ad1 = jnp.sum(h1 * adstw[1][None, :], axis=1)
    es0 = jnp.exp(_leaky(as0 + ad0))
    es1 = jnp.exp(_leaky(as1 + ad1))
    accinit = jnp.concatenate([h0 * es0[:, None], h1 * es1[:, None]], axis=1)
    accinit_ref[...] = jnp.stack(
        [accinit[:, g * FBLK:(g + 1) * FBLK] for g in range(H * NF)], axis=0)
    col0 = jnp.stack([es0, es1], axis=0)[:, :, None]
    deninit_ref[...] = jnp.concatenate(
        [col0, jnp.zeros((H, QB, 15), jnp.float32)], axis=2)
    asrcv_ref[...] = jnp.stack([as0, as1], axis=0)[:, None, :]
    adstv_ref[...] = jnp.stack([ad0, ad1], axis=0)[:, None, :]


def _fused(prototypes, W, att_src, att_dst):
    return pl.pallas_call(
        _fused_body,
        grid=(N // QB,),
        in_specs=[
            pl.BlockSpec((QB, D), lambda i: (i, 0)),
            pl.BlockSpec((N, D), lambda i: (0, 0)),
            pl.BlockSpec((D, H * C), lambda i: (0, 0)),
            pl.BlockSpec((H, C), lambda i: (0, 0)),
            pl.BlockSpec((H, C), lambda i: (0, 0)),
        ],
        out_specs=[
            pl.BlockSpec((8, QB), lambda i: (0, i)),
            pl.BlockSpec((H * NF, QB, FBLK), lambda i: (0, i, 0)),
            pl.BlockSpec((H * NF, QB, FBLK), lambda i: (0, i, 0)),
            pl.BlockSpec((H, QB, 16), lambda i: (0, i, 0)),
            pl.BlockSpec((H, 1, QB), lambda i: (0, 0, i)),
            pl.BlockSpec((H, 1, QB), lambda i: (0, 0, i)),
        ],
        out_shape=[
            jax.ShapeDtypeStruct((8, N), jnp.int32),
            jax.ShapeDtypeStruct((H * NF, N, FBLK), jnp.float32),
            jax.ShapeDtypeStruct((H * NF, N, FBLK), jnp.float32),
            jax.ShapeDtypeStruct((H, N, 16), jnp.float32),
            jax.ShapeDtypeStruct((H, 1, N), jnp.float32),
            jax.ShapeDtypeStruct((H, 1, N), jnp.float32),
        ],
    )(prototypes, prototypes, W, att_src, att_dst)


def _sc_body(xw_hbm, accinit_hbm, deninit_hbm, asrc_hbm, adst_hbm,
             d0_hbm, d1_hbm, d2_hbm,
             acc_out, den_out,
             acc_s, den_s,
             xw_v, msgs_v, denrows_v, ex0_v, ex1_v, ex2_v,
             db0_v, db1_v, db2_v, asrc_v, adst_v, idxc_v):
    exs_v = (ex0_v, ex1_v, ex2_v)
    dbs_v = (db0_v, db1_v, db2_v)
    c = lax.axis_index("c")
    s = lax.axis_index("s")
    row0 = s * RPT
    lanes = lax.iota(jnp.int32, 16)
    zeros16 = jnp.zeros((16,), jnp.float32)
    izeros16 = jnp.zeros((16,), jnp.int32)

    pltpu.sync_copy(adst_hbm.at[c, 0], adst_v)
    pltpu.sync_copy(asrc_hbm.at[c, 0, pl.ds(row0, RPT)], asrc_v)
    for k, dbuf in enumerate((d0_hbm, d1_hbm, d2_hbm)):
        pltpu.sync_copy(dbuf.at[pl.ds(row0, RPT)], dbs_v[k])

    for k in range(3):
        for i in range(RPT // 16):
            dk = dbs_v[k][pl.ds(i * 16, 16)]
            adst_g = plsc.load_gather(adst_v, [dk])
            alpha = asrc_v[pl.ds(i * 16, 16)] + adst_g
            ex = jnp.exp(jnp.where(alpha >= 0, alpha, 0.2 * alpha))
            rowid = row0 + i * 16 + lanes
            exs_v[k][pl.ds(i * 16, 16)] = jnp.where(dk == rowid, 0.0, ex)

    for e in range(3 * RPC):
        denrows_v[e, :] = zeros16

    for f in range(NF):
        g = c * NF + f
        pltpu.sync_copy(accinit_hbm.at[g, pl.ds(row0, RPT), :],
                        acc_s.at[pl.ds(row0, RPT), :])
        if f == 0:
            pltpu.sync_copy(deninit_hbm.at[c, pl.ds(row0, RPT), :],
                            den_s.at[pl.ds(row0, RPT), :])
        pltpu.sync_copy(xw_hbm.at[g, pl.ds(row0, RPT), :], xw_v)
        plsc.subcore_barrier()

        def chunk(j, _):
            jr = j * RPC
            for k in range(3):
                for i in range(RPC // 16):
                    idxc_v[pl.ds(k * RPC + i * 16, 16)] = \
                        dbs_v[k][pl.ds(jr + i * 16, 16)]
            for r in range(RPC):
                xwb = [xw_v[jr + r, pl.ds(bb * 16, 16)]
                       for bb in range(FBLK // 16)]
                for k in range(3):
                    e = k * RPC + r
                    exb = plsc.load_gather(
                        exs_v[k], [jnp.full((16,), jr + r, jnp.int32)])
                    for bb in range(FBLK // 16):
                        msgs_v[e, pl.ds(bb * 16, 16)] = xwb[bb] * exb
            if f == 0:
                for k in range(3):
                    for i in range(RPC // 16):
                        e0 = k * RPC + i * 16
                        evec = e0 + lanes
                        exv = exs_v[k][pl.ds(jr + i * 16, 16)]
                        plsc.store_scatter(denrows_v, [evec, izeros16],
                                           exv)
                pltpu.sync_copy(denrows_v, den_s.at[idxc_v], add=True)
            pltpu.sync_copy(msgs_v, acc_s.at[idxc_v], add=True)
            return 0

        lax.fori_loop(0, NCH, chunk, 0)
        plsc.subcore_barrier()
        pltpu.sync_copy(acc_s.at[pl.ds(row0, RPT), :],
                        acc_out.at[c, f, pl.ds(row0, RPT), :])
        if f == 0:
            pltpu.sync_copy(den_s.at[pl.ds(row0, RPT), :],
                            den_out.at[c, pl.ds(row0, RPT), :])
        plsc.subcore_barrier()


@functools.lru_cache(maxsize=1)
def _sc_kernel():
    return functools.partial(
        pl.kernel,
        out_type=[
        jax.ShapeDtypeStruct((H, NF, N, FBLK), jnp.float32),
        jax.ShapeDtypeStruct((H, N, 16), jnp.float32),
    ],
        mesh=plsc.VectorSubcoreMesh(core_axis_name="c", subcore_axis_name="s"),
        compiler_params=pltpu.CompilerParams(needs_layout_passes=False),
        scratch_types=[
        pltpu.VMEM_SHARED((N, FBLK), jnp.float32),
            pltpu.VMEM_SHARED((N, 16), jnp.float32),
            pltpu.VMEM((RPT, FBLK), jnp.float32),
            pltpu.VMEM((3 * RPC, FBLK), jnp.float32),
            pltpu.VMEM((3 * RPC, 16), jnp.float32),
            pltpu.VMEM((RPT,), jnp.float32),
            pltpu.VMEM((RPT,), jnp.float32),
            pltpu.VMEM((RPT,), jnp.float32),
            pltpu.VMEM((RPT,), jnp.int32),
            pltpu.VMEM((RPT,), jnp.int32),
            pltpu.VMEM((RPT,), jnp.int32),
            pltpu.VMEM((RPT,), jnp.float32),
            pltpu.VMEM((N,), jnp.float32),
            pltpu.VMEM((3 * RPC,), jnp.int32),
        ],
    )(_sc_body)


def _sc_scatter(*args):
    return _sc_kernel()(*args)


def _post_body(acc_ref, d0_ref, d1_ref, p_ref, bias_ref, gamma_ref, beta_ref,
               out_ref):
    acc = acc_ref[...]
    d0 = d0_ref[...]
    d1 = d1_ref[...]
    a0 = jnp.concatenate([acc[0, f] for f in range(NF)], axis=1)
    a1 = jnp.concatenate([acc[1, f] for f in range(NF)], axis=1)
    out = 0.5 * (a0 / d0 + a1 / d1) + bias_ref[...]
    mu = jnp.mean(out, axis=1, keepdims=True)
    var = jnp.mean((out - mu) ** 2, axis=1, keepdims=True)
    out = (out - mu) / jnp.sqrt(var + 1e-5) * gamma_ref[...] + beta_ref[...]
    out_ref[...] = p_ref[...] + jnp.maximum(out, 0.0)


def _post(acc, den, prototypes, bias, gamma, beta):
    return pl.pallas_call(
        _post_body,
        grid=(N // QB,),
        in_specs=[
            pl.BlockSpec((H, NF, QB, FBLK), lambda i: (0, 0, i, 0)),
            pl.BlockSpec((QB, 1), lambda i: (i, 0)),
            pl.BlockSpec((QB, 1), lambda i: (i, 0)),
            pl.BlockSpec((QB, C), lambda i: (i, 0)),
            pl.BlockSpec((1, C), lambda i: (0, 0)),
            pl.BlockSpec((1, C), lambda i: (0, 0)),
            pl.BlockSpec((1, C), lambda i: (0, 0)),
        ],
        out_specs=pl.BlockSpec((QB, C), lambda i: (i, 0)),
        out_shape=jax.ShapeDtypeStruct((N, C), jnp.float32),
    )(acc, den[0, :, :1], den[1, :, :1], prototypes,
      bias.reshape(1, C), gamma.reshape(1, C), beta.reshape(1, C))


def kernel(prototypes, labels, W, att_src, att_dst, bias, gamma, beta):
    idx8, xw, accinit, deninit, asrcv, adstv = _fused(
        prototypes, W, att_src, att_dst)
    acc, den = _sc_scatter(xw, accinit, deninit, asrcv, adstv,
                           idx8[0], idx8[1], idx8[2])
    return _post(acc, den, prototypes, bias, gamma, beta)


# speedup vs baseline: 1.2379x; 1.1873x over previous
import functools

import jax
import jax.numpy as jnp
from jax import lax
from jax.experimental import pallas as pl
from jax.experimental.pallas import tpu as pltpu
from jax.experimental.pallas import tpu_sc as plsc

N = 8192
D = 256
H = 2
C = 256

QB = 512
KB = 512

RPT = 512
RPC = 32
NCH = RPT // RPC
FBLK = 32
DSH = 5
DROWS = N // FBLK
NF = C // FBLK


def _top3_body(q_ref, k_ref, idx_ref):
    q = q_ref[...]
    qsq = jnp.sum(q * q, axis=1)

    def step(t, carry):
        v1, v2, v3, i1, i2, i3 = carry
        k = k_ref[pl.ds(t * KB, KB), :]
        ksq = jnp.sum(k * k, axis=1)
        dot = jax.lax.dot_general(
            q, k, (((1,), (1,)), ((), ())),
            preferred_element_type=jnp.float32)
        d2 = qsq[:, None] + ksq[None, :] - 2.0 * dot
        dist = jnp.sqrt(jnp.maximum(d2, 0.0))
        col = jax.lax.broadcasted_iota(jnp.int32, (QB, KB), 1) + t * KB

        def block_min(dmat):
            m = jnp.min(dmat, axis=1)
            idx = jnp.min(jnp.where(dmat == m[:, None], col, N), axis=1)
            dmat2 = jnp.where(col == idx[:, None], jnp.inf, dmat)
            return m, idx, dmat2

        m1, j1, dist = block_min(dist)
        m2, j2, dist = block_min(dist)
        m3, j3, dist = block_min(dist)

        def insert(m, j, v1, v2, v3, i1, i2, i3):
            c1 = m < v1
            c2 = m < v2
            c3 = m < v3
            nv3 = jnp.where(c3, jnp.where(c2, v2, m), v3)
            ni3 = jnp.where(c3, jnp.where(c2, i2, j), i3)
            nv2 = jnp.where(c2, jnp.where(c1, v1, m), v2)
            ni2 = jnp.where(c2, jnp.where(c1, i1, j), i2)
            nv1 = jnp.where(c1, m, v1)
            ni1 = jnp.where(c1, j, i1)
            return nv1, nv2, nv3, ni1, ni2, ni3

        v1, v2, v3, i1, i2, i3 = insert(m1, j1, v1, v2, v3, i1, i2, i3)
        v1, v2, v3, i1, i2, i3 = insert(m2, j2, v1, v2, v3, i1, i2, i3)
        v1, v2, v3, i1, i2, i3 = insert(m3, j3, v1, v2, v3, i1, i2, i3)
        return v1, v2, v3, i1, i2, i3

    inf = jnp.full((QB,), jnp.inf, dtype=jnp.float32)
    zero = jnp.zeros((QB,), dtype=jnp.int32)
    v1, v2, v3, i1, i2, i3 = jax.lax.fori_loop(
        0, N // KB, step, (inf, inf, inf, zero, zero, zero))
    idx_ref[...] = jnp.stack([i1, i2, i3, i1, i1, i1, i1, i1], axis=0)


def _leaky(x):
    return jnp.where(x >= 0, x, 0.2 * x)


def _fused_body(p_ref, k_ref, w_ref, asrcw_ref, adstw_ref,
                idx_ref, xw_ref, accinit_ref, deninit_ref,
                asrcv_ref, adstv_ref):
    _top3_body(p_ref, k_ref, idx_ref)
    p = p_ref[...]
    w = w_ref[...]
    xw = jnp.dot(p, w, preferred_element_type=jnp.float32)
    xw_ref[...] = jnp.stack(
        [xw[:, g * FBLK:(g + 1) * FBLK] for g in range(H * NF)], axis=0)
    asrcw = asrcw_ref[...]
    adstw = adstw_ref[...]
    h0 = xw[:, :C]
    h1 = xw[:, C:]
    as0 = jnp.sum(h0 * asrcw[0][None, :], axis=1)
    as1 = jnp.sum(h1 * asrcw[1][None, :], axis=1)
    ad0 = jnp.sum(h0 * adstw[0][None, :], axis=1)
    ad1 = jnp.sum(h1 * adstw[1][None, :], axis=1)
    es0 = jnp.exp(_leaky(as0 + ad0))
    es1 = jnp.exp(_leaky(as1 + ad1))
    accinit = jnp.concatenate([h0 * es0[:, None], h1 * es1[:, None]], axis=1)
    accinit_ref[...] = jnp.stack(
        [accinit[:, g * FBLK:(g + 1) * FBLK] for g in range(H * NF)], axis=0)
    col0 = jnp.stack([es0, es1], axis=0)[:, :, None]
    deninit_ref[...] = jnp.concatenate(
        [col0, jnp.zeros((H, QB, 15), jnp.float32)], axis=2)
    asrcv_ref[...] = jnp.stack([as0, as1], axis=0)[:, None, :]
    adstv_ref[...] = jnp.stack([ad0, ad1], axis=0)[:, None, :]


def _fused(prototypes, W, att_src, att_dst):
    return pl.pallas_call(
        _fused_body,
        grid=(N // QB,),
        in_specs=[
            pl.BlockSpec((QB, D), lambda i: (i, 0)),
            pl.BlockSpec((N, D), lambda i: (0, 0)),
            pl.BlockSpec((D, H * C), lambda i: (0, 0)),
            pl.BlockSpec((H, C), lambda i: (0, 0)),
            pl.BlockSpec((H, C), lambda i: (0, 0)),
        ],
        out_specs=[
            pl.BlockSpec((8, QB), lambda i: (0, i)),
            pl.BlockSpec((H * NF, QB, FBLK), lambda i: (0, i, 0)),
            pl.BlockSpec((H * NF, QB, FBLK), lambda i: (0, i, 0)),
            pl.BlockSpec((H, QB, 16), lambda i: (0, i, 0)),
            pl.BlockSpec((H, 1, QB), lambda i: (0, 0, i)),
            pl.BlockSpec((H, 1, QB), lambda i: (0, 0, i)),
        ],
        out_shape=[
            jax.ShapeDtypeStruct((8, N), jnp.int32),
            jax.ShapeDtypeStruct((H * NF, N, FBLK), jnp.float32),
            jax.ShapeDtypeStruct((H * NF, N, FBLK), jnp.float32),
            jax.ShapeDtypeStruct((H, N, 16), jnp.float32),
            jax.ShapeDtypeStruct((H, 1, N), jnp.float32),
            jax.ShapeDtypeStruct((H, 1, N), jnp.float32),
        ],
    )(prototypes, prototypes, W, att_src, att_dst)


def _sc_body(xw_hbm, accinit_hbm, deninit_hbm, asrc_hbm, adst_hbm,
             d0_hbm, d1_hbm, d2_hbm,
             acc_out, den_out,
             acc_s, den_s,
             xw_v, msgs_v, denrows_v, ex0_v, ex1_v, ex2_v,
             db0_v, db1_v, db2_v, asrc_v, adst_v, idxc_v):
    exs_v = (ex0_v, ex1_v, ex2_v)
    dbs_v = (db0_v, db1_v, db2_v)
    c = lax.axis_index("c")
    s = lax.axis_index("s")
    row0 = s * RPT
    lanes = lax.iota(jnp.int32, 16)
    zeros16 = jnp.zeros((16,), jnp.float32)
    izeros16 = jnp.zeros((16,), jnp.int32)

    pltpu.sync_copy(adst_hbm.at[c, 0], adst_v)
    pltpu.sync_copy(asrc_hbm.at[c, 0, pl.ds(row0, RPT)], asrc_v)
    for k, dbuf in enumerate((d0_hbm, d1_hbm, d2_hbm)):
        pltpu.sync_copy(dbuf.at[pl.ds(row0, RPT)], dbs_v[k])

    for k in range(3):
        for i in range(RPT // 16):
            dk = dbs_v[k][pl.ds(i * 16, 16)]
            adst_g = plsc.load_gather(adst_v, [dk])
            alpha = asrc_v[pl.ds(i * 16, 16)] + adst_g
            ex = jnp.exp(jnp.where(alpha >= 0, alpha, 0.2 * alpha))
            rowid = row0 + i * 16 + lanes
            exs_v[k][pl.ds(i * 16, 16)] = jnp.where(dk == rowid, 0.0, ex)

    for e in range(3 * RPC):
        denrows_v[e, :] = zeros16

    for f in range(NF):
        g = c * NF + f
        pltpu.sync_copy(accinit_hbm.at[g, pl.ds(row0, RPT), :],
                        acc_s.at[pl.ds(row0, RPT), :])
        if f == 0:
            pltpu.sync_copy(deninit_hbm.at[c, pl.ds(row0, RPT), :],
                            den_s.at[pl.ds(row0, RPT), :])
        pltpu.sync_copy(xw_hbm.at[g, pl.ds(row0, RPT), :], xw_v)
        plsc.subcore_barrier()

        def chunk(j, _):
            jr = j * RPC
            for k in range(3):
                for i in range(RPC // 16):
                    idxc_v[pl.ds(k * RPC + i * 16, 16)] = \
                        dbs_v[k][pl.ds(jr + i * 16, 16)]
            for r in range(RPC):
                xwb = [xw_v[jr + r, pl.ds(bb * 16, 16)]
                       for bb in range(FBLK // 16)]
                for k in range(3):
                    e = k * RPC + r
                    exb = plsc.load_gather(
                        exs_v[k], [jnp.full((16,), jr + r, jnp.int32)])
                    for bb in range(FBLK // 16):
                        msgs_v[e, pl.ds(bb * 16, 16)] = xwb[bb] * exb
            if f == 0:
                for k in range(3):
                    for i in range(RPC // 16):
                        e0 = k * RPC + i * 16
                        evec = e0 + lanes
                        exv = exs_v[k][pl.ds(jr + i * 16, 16)]
                        plsc.store_scatter(denrows_v, [evec, izeros16],
                                           exv)
                pltpu.sync_copy(denrows_v, den_s.at[idxc_v], add=True)
            pltpu.sync_copy(msgs_v, acc_s.at[idxc_v], add=True)
            return 0

        lax.fori_loop(0, NCH, chunk, 0)
        plsc.subcore_barrier()
        pltpu.sync_copy(acc_s.at[pl.ds(row0, RPT), :],
                        acc_out.at[c, f, pl.ds(row0, RPT), :])
        if f == 0:
            pltpu.sync_copy(den_s.at[pl.ds(row0, RPT), :],
                            den_out.at[c, pl.ds(row0, RPT), :])
        plsc.subcore_barrier()


@functools.lru_cache(maxsize=1)
def _sc_kernel():
    return functools.partial(
        pl.kernel,
        out_type=[
        jax.ShapeDtypeStruct((H, NF, N, FBLK), jnp.float32),
        jax.ShapeDtypeStruct((H, N, 16), jnp.float32),
    ],
        mesh=plsc.VectorSubcoreMesh(core_axis_name="c", subcore_axis_name="s"),
        compiler_params=pltpu.CompilerParams(needs_layout_passes=False),
        scratch_types=[
        pltpu.VMEM_SHARED((N, FBLK), jnp.float32),
            pltpu.VMEM_SHARED((N, 16), jnp.float32),
            pltpu.VMEM((RPT, FBLK), jnp.float32),
            pltpu.VMEM((3 * RPC, FBLK), jnp.float32),
            pltpu.VMEM((3 * RPC, 16), jnp.float32),
            pltpu.VMEM((RPT,), jnp.float32),
            pltpu.VMEM((RPT,), jnp.float32),
            pltpu.VMEM((RPT,), jnp.float32),
            pltpu.VMEM((RPT,), jnp.int32),
            pltpu.VMEM((RPT,), jnp.int32),
            pltpu.VMEM((RPT,), jnp.int32),
            pltpu.VMEM((RPT,), jnp.float32),
            pltpu.VMEM((N,), jnp.float32),
            pltpu.VMEM((3 * RPC,), jnp.int32),
        ],
    )(_sc_body)


def _sc_scatter(*args):
    return _sc_kernel()(*args)


def _post_body(acc_ref, d0_ref, d1_ref, p_ref, bias_ref, gamma_ref, beta_ref,
               out_ref):
    acc = acc_ref[...]
    d0 = d0_ref[...]
    d1 = d1_ref[...]
    a0 = jnp.concatenate([acc[0, f] for f in range(NF)], axis=1)
    a1 = jnp.concatenate([acc[1, f] for f in range(NF)], axis=1)
    out = 0.5 * (a0 / d0 + a1 / d1) + bias_ref[...]
    mu = jnp.mean(out, axis=1, keepdims=True)
    var = jnp.mean((out - mu) ** 2, axis=1, keepdims=True)
    out = (out - mu) / jnp.sqrt(var + 1e-5) * gamma_ref[...] + beta_ref[...]
    out_ref[...] = p_ref[...] + jnp.maximum(out, 0.0)


def _post(acc, den, prototypes, bias, gamma, beta):
    return pl.pallas_call(
        _post_body,
        grid=(N // QB,),
        in_specs=[
            pl.BlockSpec((H, NF, QB, FBLK), lambda i: (0, 0, i, 0)),
            pl.BlockSpec((QB, 1), lambda i: (i, 0)),
            pl.BlockSpec((QB, 1), lambda i: (i, 0)),
            pl.BlockSpec((QB, C), lambda i: (i, 0)),
            pl.BlockSpec((1, C), lambda i: (0, 0)),
            pl.BlockSpec((1, C), lambda i: (0, 0)),
            pl.BlockSpec((1, C), lambda i: (0, 0)),
        ],
        out_specs=pl.BlockSpec((QB, C), lambda i: (i, 0)),
        out_shape=jax.ShapeDtypeStruct((N, C), jnp.float32),
    )(acc, den[0, :, :1], den[1, :, :1], prototypes,
      bias.reshape(1, C), gamma.reshape(1, C), beta.reshape(1, C))


def kernel(prototypes, labels, W, att_src, att_dst, bias, gamma, beta):
    idx8, xw, accinit, deninit, asrcv, adstv = _fused(
        prototypes, W, att_src, att_dst)
    acc, den = _sc_scatter(xw, accinit, deninit, asrcv, adstv,
                           idx8[0], idx8[1], idx8[2])
    return _post(acc, den, prototypes, bias, gamma, beta)


# speedup vs baseline: 1.4507x; 1.1719x over previous
import functools

import jax
import jax.numpy as jnp
from jax import lax
from jax.experimental import pallas as pl
from jax.experimental.pallas import tpu as pltpu
from jax.experimental.pallas import tpu_sc as plsc

N = 8192
D = 256
H = 2
C = 256

QB = 512
KB = 1024

RPT = 512
RPC = 32
NCH = RPT // RPC
FBLK = 32
DSH = 5
DROWS = N // FBLK
NF = C // FBLK


def _top3_body(q_ref, k_ref, idx_ref):
    q = q_ref[...]
    qsq = jnp.sum(q * q, axis=1)

    def step(t, carry):
        v1, v2, v3, i1, i2, i3 = carry
        k = k_ref[pl.ds(t * KB, KB), :]
        ksq = jnp.sum(k * k, axis=1)
        dot = jax.lax.dot_general(
            q, k, (((1,), (1,)), ((), ())),
            preferred_element_type=jnp.float32)
        d2 = qsq[:, None] + ksq[None, :] - 2.0 * dot
        dist = jnp.sqrt(jnp.maximum(d2, 0.0))
        col = jax.lax.broadcasted_iota(jnp.int32, (QB, KB), 1) + t * KB

        def block_min(dmat):
            m = jnp.min(dmat, axis=1)
            idx = jnp.min(jnp.where(dmat == m[:, None], col, N), axis=1)
            dmat2 = jnp.where(col == idx[:, None], jnp.inf, dmat)
            return m, idx, dmat2

        m1, j1, dist = block_min(dist)
        m2, j2, dist = block_min(dist)
        m3, j3, dist = block_min(dist)

        def insert(m, j, v1, v2, v3, i1, i2, i3):
            c1 = m < v1
            c2 = m < v2
            c3 = m < v3
            nv3 = jnp.where(c3, jnp.where(c2, v2, m), v3)
            ni3 = jnp.where(c3, jnp.where(c2, i2, j), i3)
            nv2 = jnp.where(c2, jnp.where(c1, v1, m), v2)
            ni2 = jnp.where(c2, jnp.where(c1, i1, j), i2)
            nv1 = jnp.where(c1, m, v1)
            ni1 = jnp.where(c1, j, i1)
            return nv1, nv2, nv3, ni1, ni2, ni3

        v1, v2, v3, i1, i2, i3 = insert(m1, j1, v1, v2, v3, i1, i2, i3)
        v1, v2, v3, i1, i2, i3 = insert(m2, j2, v1, v2, v3, i1, i2, i3)
        v1, v2, v3, i1, i2, i3 = insert(m3, j3, v1, v2, v3, i1, i2, i3)
        return v1, v2, v3, i1, i2, i3

    inf = jnp.full((QB,), jnp.inf, dtype=jnp.float32)
    zero = jnp.zeros((QB,), dtype=jnp.int32)
    v1, v2, v3, i1, i2, i3 = jax.lax.fori_loop(
        0, N // KB, step, (inf, inf, inf, zero, zero, zero))
    idx_ref[...] = jnp.stack([i1, i2, i3, i1, i1, i1, i1, i1], axis=0)


def _leaky(x):
    return jnp.where(x >= 0, x, 0.2 * x)


def _fused_body(p_ref, k_ref, w_ref, asrcw_ref, adstw_ref,
                idx_ref, xw_ref, accinit_ref, deninit_ref,
                asrcv_ref, adstv_ref):
    _top3_body(p_ref, k_ref, idx_ref)
    p = p_ref[...]
    w = w_ref[...]
    xw = jnp.dot(p, w, preferred_element_type=jnp.float32)
    xw_ref[...] = jnp.stack(
        [xw[:, g * FBLK:(g + 1) * FBLK] for g in range(H * NF)], axis=0)
    asrcw = asrcw_ref[...]
    adstw = adstw_ref[...]
    h0 = xw[:, :C]
    h1 = xw[:, C:]
    as0 = jnp.sum(h0 * asrcw[0][None, :], axis=1)
    as1 = jnp.sum(h1 * asrcw[1][None, :], axis=1)
    ad0 = jnp.sum(h0 * adstw[0][None, :], axis=1)
    ad1 = jnp.sum(h1 * adstw[1][None, :], axis=1)
    es0 = jnp.exp(_leaky(as0 + ad0))
    es1 = jnp.exp(_leaky(as1 + ad1))
    accinit = jnp.concatenate([h0 * es0[:, None], h1 * es1[:, None]], axis=1)
    accinit_ref[...] = jnp.stack(
        [accinit[:, g * FBLK:(g + 1) * FBLK] for g in range(H * NF)], axis=0)
    col0 = jnp.stack([es0, es1], axis=0)[:, :, None]
    deninit_ref[...] = jnp.concatenate(
        [col0, jnp.zeros((H, QB, 15), jnp.float32)], axis=2)
    asrcv_ref[...] = jnp.stack([as0, as1], axis=0)[:, None, :]
    adstv_ref[...] = jnp.stack([ad0, ad1], axis=0)[:, None, :]


def _fused(prototypes, W, att_src, att_dst):
    return pl.pallas_call(
        _fused_body,
        grid=(N // QB,),
        in_specs=[
            pl.BlockSpec((QB, D), lambda i: (i, 0)),
            pl.BlockSpec((N, D), lambda i: (0, 0)),
            pl.BlockSpec((D, H * C), lambda i: (0, 0)),
            pl.BlockSpec((H, C), lambda i: (0, 0)),
            pl.BlockSpec((H, C), lambda i: (0, 0)),
        ],
        out_specs=[
            pl.BlockSpec((8, QB), lambda i: (0, i)),
            pl.BlockSpec((H * NF, QB, FBLK), lambda i: (0, i, 0)),
            pl.BlockSpec((H * NF, QB, FBLK), lambda i: (0, i, 0)),
            pl.BlockSpec((H, QB, 16), lambda i: (0, i, 0)),
            pl.BlockSpec((H, 1, QB), lambda i: (0, 0, i)),
            pl.BlockSpec((H, 1, QB), lambda i: (0, 0, i)),
        ],
        out_shape=[
            jax.ShapeDtypeStruct((8, N), jnp.int32),
            jax.ShapeDtypeStruct((H * NF, N, FBLK), jnp.float32),
            jax.ShapeDtypeStruct((H * NF, N, FBLK), jnp.float32),
            jax.ShapeDtypeStruct((H, N, 16), jnp.float32),
            jax.ShapeDtypeStruct((H, 1, N), jnp.float32),
            jax.ShapeDtypeStruct((H, 1, N), jnp.float32),
        ],
    )(prototypes, prototypes, W, att_src, att_dst)


def _sc_body(xw_hbm, accinit_hbm, deninit_hbm, asrc_hbm, adst_hbm,
             d0_hbm, d1_hbm, d2_hbm,
             acc_out, den_out,
             acc_s, den_s,
             xw_v, msgs_v, denrows_v, ex0_v, ex1_v, ex2_v,
             db0_v, db1_v, db2_v, asrc_v, adst_v, idxc_v):
    exs_v = (ex0_v, ex1_v, ex2_v)
    dbs_v = (db0_v, db1_v, db2_v)
    c = lax.axis_index("c")
    s = lax.axis_index("s")
    row0 = s * RPT
    lanes = lax.iota(jnp.int32, 16)
    zeros16 = jnp.zeros((16,), jnp.float32)
    izeros16 = jnp.zeros((16,), jnp.int32)

    pltpu.sync_copy(adst_hbm.at[c, 0], adst_v)
    pltpu.sync_copy(asrc_hbm.at[c, 0, pl.ds(row0, RPT)], asrc_v)
    for k, dbuf in enumerate((d0_hbm, d1_hbm, d2_hbm)):
        pltpu.sync_copy(dbuf.at[pl.ds(row0, RPT)], dbs_v[k])

    for k in range(3):
        for i in range(RPT // 16):
            dk = dbs_v[k][pl.ds(i * 16, 16)]
            adst_g = plsc.load_gather(adst_v, [dk])
            alpha = asrc_v[pl.ds(i * 16, 16)] + adst_g
            ex = jnp.exp(jnp.where(alpha >= 0, alpha, 0.2 * alpha))
            rowid = row0 + i * 16 + lanes
            exs_v[k][pl.ds(i * 16, 16)] = jnp.where(dk == rowid, 0.0, ex)

    for e in range(3 * RPC):
        denrows_v[e, :] = zeros16

    for f in range(NF):
        g = c * NF + f
        pltpu.sync_copy(accinit_hbm.at[g, pl.ds(row0, RPT), :],
                        acc_s.at[pl.ds(row0, RPT), :])
        if f == 0:
            pltpu.sync_copy(deninit_hbm.at[c, pl.ds(row0, RPT), :],
                            den_s.at[pl.ds(row0, RPT), :])
        pltpu.sync_copy(xw_hbm.at[g, pl.ds(row0, RPT), :], xw_v)
        plsc.subcore_barrier()

        def chunk(j, _):
            jr = j * RPC
            for k in range(3):
                for i in range(RPC // 16):
                    idxc_v[pl.ds(k * RPC + i * 16, 16)] = \
                        dbs_v[k][pl.ds(jr + i * 16, 16)]
            for r in range(RPC):
                xwb = [xw_v[jr + r, pl.ds(bb * 16, 16)]
                       for bb in range(FBLK // 16)]
                for k in range(3):
                    e = k * RPC + r
                    exb = plsc.load_gather(
                        exs_v[k], [jnp.full((16,), jr + r, jnp.int32)])
                    for bb in range(FBLK // 16):
                        msgs_v[e, pl.ds(bb * 16, 16)] = xwb[bb] * exb
            if f == 0:
                for k in range(3):
                    for i in range(RPC // 16):
                        e0 = k * RPC + i * 16
                        evec = e0 + lanes
                        exv = exs_v[k][pl.ds(jr + i * 16, 16)]
                        plsc.store_scatter(denrows_v, [evec, izeros16],
                                           exv)
                pltpu.sync_copy(denrows_v, den_s.at[idxc_v], add=True)
            pltpu.sync_copy(msgs_v, acc_s.at[idxc_v], add=True)
            return 0

        lax.fori_loop(0, NCH, chunk, 0)
        plsc.subcore_barrier()
        pltpu.sync_copy(acc_s.at[pl.ds(row0, RPT), :],
                        acc_out.at[c, f, pl.ds(row0, RPT), :])
        if f == 0:
            pltpu.sync_copy(den_s.at[pl.ds(row0, RPT), :],
                            den_out.at[c, pl.ds(row0, RPT), :])
        plsc.subcore_barrier()


@functools.lru_cache(maxsize=1)
def _sc_kernel():
    return functools.partial(
        pl.kernel,
        out_type=[
        jax.ShapeDtypeStruct((H, NF, N, FBLK), jnp.float32),
        jax.ShapeDtypeStruct((H, N, 16), jnp.float32),
    ],
        mesh=plsc.VectorSubcoreMesh(core_axis_name="c", subcore_axis_name="s"),
        compiler_params=pltpu.CompilerParams(needs_layout_passes=False),
        scratch_types=[
        pltpu.VMEM_SHARED((N, FBLK), jnp.float32),
            pltpu.VMEM_SHARED((N, 16), jnp.float32),
            pltpu.VMEM((RPT, FBLK), jnp.float32),
            pltpu.VMEM((3 * RPC, FBLK), jnp.float32),
            pltpu.VMEM((3 * RPC, 16), jnp.float32),
            pltpu.VMEM((RPT,), jnp.float32),
            pltpu.VMEM((RPT,), jnp.float32),
            pltpu.VMEM((RPT,), jnp.float32),
            pltpu.VMEM((RPT,), jnp.int32),
            pltpu.VMEM((RPT,), jnp.int32),
            pltpu.VMEM((RPT,), jnp.int32),
            pltpu.VMEM((RPT,), jnp.float32),
            pltpu.VMEM((N,), jnp.float32),
            pltpu.VMEM((3 * RPC,), jnp.int32),
        ],
    )(_sc_body)


def _sc_scatter(*args):
    return _sc_kernel()(*args)


def _post_body(acc_ref, d0_ref, d1_ref, p_ref, bias_ref, gamma_ref, beta_ref,
               out_ref):
    acc = acc_ref[...]
    d0 = d0_ref[...]
    d1 = d1_ref[...]
    a0 = jnp.concatenate([acc[0, f] for f in range(NF)], axis=1)
    a1 = jnp.concatenate([acc[1, f] for f in range(NF)], axis=1)
    out = 0.5 * (a0 / d0 + a1 / d1) + bias_ref[...]
    mu = jnp.mean(out, axis=1, keepdims=True)
    var = jnp.mean((out - mu) ** 2, axis=1, keepdims=True)
    out = (out - mu) / jnp.sqrt(var + 1e-5) * gamma_ref[...] + beta_ref[...]
    out_ref[...] = p_ref[...] + jnp.maximum(out, 0.0)


def _post(acc, den, prototypes, bias, gamma, beta):
    return pl.pallas_call(
        _post_body,
        grid=(N // QB,),
        in_specs=[
            pl.BlockSpec((H, NF, QB, FBLK), lambda i: (0, 0, i, 0)),
            pl.BlockSpec((QB, 1), lambda i: (i, 0)),
            pl.BlockSpec((QB, 1), lambda i: (i, 0)),
            pl.BlockSpec((QB, C), lambda i: (i, 0)),
            pl.BlockSpec((1, C), lambda i: (0, 0)),
            pl.BlockSpec((1, C), lambda i: (0, 0)),
            pl.BlockSpec((1, C), lambda i: (0, 0)),
        ],
        out_specs=pl.BlockSpec((QB, C), lambda i: (i, 0)),
        out_shape=jax.ShapeDtypeStruct((N, C), jnp.float32),
    )(acc, den[0, :, :1], den[1, :, :1], prototypes,
      bias.reshape(1, C), gamma.reshape(1, C), beta.reshape(1, C))


def kernel(prototypes, labels, W, att_src, att_dst, bias, gamma, beta):
    idx8, xw, accinit, deninit, asrcv, adstv = _fused(
        prototypes, W, att_src, att_dst)
    acc, den = _sc_scatter(xw, accinit, deninit, asrcv, adstv,
                           idx8[0], idx8[1], idx8[2])
    return _post(acc, den, prototypes, bias, gamma, beta)


# speedup vs baseline: 1.5345x; 1.0578x over previous
import functools

import jax
import jax.numpy as jnp
from jax import lax
from jax.experimental import pallas as pl
from jax.experimental.pallas import tpu as pltpu
from jax.experimental.pallas import tpu_sc as plsc

N = 8192
D = 256
H = 2
C = 256

QB = 512
KB = 2048

RPT = 512
RPC = 32
NCH = RPT // RPC
FBLK = 32
DSH = 5
DROWS = N // FBLK
NF = C // FBLK


def _top3_body(q_ref, k_ref, idx_ref):
    q = q_ref[...]
    qsq = jnp.sum(q * q, axis=1)

    def step(t, carry):
        v1, v2, v3, i1, i2, i3 = carry
        k = k_ref[pl.ds(t * KB, KB), :]
        ksq = jnp.sum(k * k, axis=1)
        dot = jax.lax.dot_general(
            q, k, (((1,), (1,)), ((), ())),
            preferred_element_type=jnp.float32)
        d2 = qsq[:, None] + ksq[None, :] - 2.0 * dot
        dist = jnp.sqrt(jnp.maximum(d2, 0.0))
        col = jax.lax.broadcasted_iota(jnp.int32, (QB, KB), 1) + t * KB

        def block_min(dmat):
            m = jnp.min(dmat, axis=1)
            idx = jnp.min(jnp.where(dmat == m[:, None], col, N), axis=1)
            dmat2 = jnp.where(col == idx[:, None], jnp.inf, dmat)
            return m, idx, dmat2

        m1, j1, dist = block_min(dist)
        m2, j2, dist = block_min(dist)
        m3, j3, dist = block_min(dist)

        def insert(m, j, v1, v2, v3, i1, i2, i3):
            c1 = m < v1
            c2 = m < v2
            c3 = m < v3
            nv3 = jnp.where(c3, jnp.where(c2, v2, m), v3)
            ni3 = jnp.where(c3, jnp.where(c2, i2, j), i3)
            nv2 = jnp.where(c2, jnp.where(c1, v1, m), v2)
            ni2 = jnp.where(c2, jnp.where(c1, i1, j), i2)
            nv1 = jnp.where(c1, m, v1)
            ni1 = jnp.where(c1, j, i1)
            return nv1, nv2, nv3, ni1, ni2, ni3

        v1, v2, v3, i1, i2, i3 = insert(m1, j1, v1, v2, v3, i1, i2, i3)
        v1, v2, v3, i1, i2, i3 = insert(m2, j2, v1, v2, v3, i1, i2, i3)
        v1, v2, v3, i1, i2, i3 = insert(m3, j3, v1, v2, v3, i1, i2, i3)
        return v1, v2, v3, i1, i2, i3

    inf = jnp.full((QB,), jnp.inf, dtype=jnp.float32)
    zero = jnp.zeros((QB,), dtype=jnp.int32)
    v1, v2, v3, i1, i2, i3 = jax.lax.fori_loop(
        0, N // KB, step, (inf, inf, inf, zero, zero, zero))
    idx_ref[...] = jnp.stack([i1, i2, i3, i1, i1, i1, i1, i1], axis=0)


def _leaky(x):
    return jnp.where(x >= 0, x, 0.2 * x)


def _fused_body(p_ref, k_ref, w_ref, asrcw_ref, adstw_ref,
                idx_ref, xw_ref, accinit_ref, deninit_ref,
                asrcv_ref, adstv_ref):
    _top3_body(p_ref, k_ref, idx_ref)
    p = p_ref[...]
    w = w_ref[...]
    xw = jnp.dot(p, w, preferred_element_type=jnp.float32)
    xw_ref[...] = jnp.stack(
        [xw[:, g * FBLK:(g + 1) * FBLK] for g in range(H * NF)], axis=0)
    asrcw = asrcw_ref[...]
    adstw = adstw_ref[...]
    h0 = xw[:, :C]
    h1 = xw[:, C:]
    as0 = jnp.sum(h0 * asrcw[0][None, :], axis=1)
    as1 = jnp.sum(h1 * asrcw[1][None, :], axis=1)
    ad0 = jnp.sum(h0 * adstw[0][None, :], axis=1)
    ad1 = jnp.sum(h1 * adstw[1][None, :], axis=1)
    es0 = jnp.exp(_leaky(as0 + ad0))
    es1 = jnp.exp(_leaky(as1 + ad1))
    accinit = jnp.concatenate([h0 * es0[:, None], h1 * es1[:, None]], axis=1)
    accinit_ref[...] = jnp.stack(
        [accinit[:, g * FBLK:(g + 1) * FBLK] for g in range(H * NF)], axis=0)
    col0 = jnp.stack([es0, es1], axis=0)[:, :, None]
    deninit_ref[...] = jnp.concatenate(
        [col0, jnp.zeros((H, QB, 15), jnp.float32)], axis=2)
    asrcv_ref[...] = jnp.stack([as0, as1], axis=0)[:, None, :]
    adstv_ref[...] = jnp.stack([ad0, ad1], axis=0)[:, None, :]


def _fused(prototypes, W, att_src, att_dst):
    return pl.pallas_call(
        _fused_body,
        grid=(N // QB,),
        in_specs=[
            pl.BlockSpec((QB, D), lambda i: (i, 0)),
            pl.BlockSpec((N, D), lambda i: (0, 0)),
            pl.BlockSpec((D, H * C), lambda i: (0, 0)),
            pl.BlockSpec((H, C), lambda i: (0, 0)),
            pl.BlockSpec((H, C), lambda i: (0, 0)),
        ],
        out_specs=[
            pl.BlockSpec((8, QB), lambda i: (0, i)),
            pl.BlockSpec((H * NF, QB, FBLK), lambda i: (0, i, 0)),
            pl.BlockSpec((H * NF, QB, FBLK), lambda i: (0, i, 0)),
            pl.BlockSpec((H, QB, 16), lambda i: (0, i, 0)),
            pl.BlockSpec((H, 1, QB), lambda i: (0, 0, i)),
            pl.BlockSpec((H, 1, QB), lambda i: (0, 0, i)),
        ],
        out_shape=[
            jax.ShapeDtypeStruct((8, N), jnp.int32),
            jax.ShapeDtypeStruct((H * NF, N, FBLK), jnp.float32),
            jax.ShapeDtypeStruct((H * NF, N, FBLK), jnp.float32),
            jax.ShapeDtypeStruct((H, N, 16), jnp.float32),
            jax.ShapeDtypeStruct((H, 1, N), jnp.float32),
            jax.ShapeDtypeStruct((H, 1, N), jnp.float32),
        ],
    )(prototypes, prototypes, W, att_src, att_dst)


def _sc_body(xw_hbm, accinit_hbm, deninit_hbm, asrc_hbm, adst_hbm,
             d0_hbm, d1_hbm, d2_hbm,
             acc_out, den_out,
             acc_s, den_s,
             xw_v, msgs_v, denrows_v, ex0_v, ex1_v, ex2_v,
             db0_v, db1_v, db2_v, asrc_v, adst_v, idxc_v):
    exs_v = (ex0_v, ex1_v, ex2_v)
    dbs_v = (db0_v, db1_v, db2_v)
    c = lax.axis_index("c")
    s = lax.axis_index("s")
    row0 = s * RPT
    lanes = lax.iota(jnp.int32, 16)
    zeros16 = jnp.zeros((16,), jnp.float32)
    izeros16 = jnp.zeros((16,), jnp.int32)

    pltpu.sync_copy(adst_hbm.at[c, 0], adst_v)
    pltpu.sync_copy(asrc_hbm.at[c, 0, pl.ds(row0, RPT)], asrc_v)
    for k, dbuf in enumerate((d0_hbm, d1_hbm, d2_hbm)):
        pltpu.sync_copy(dbuf.at[pl.ds(row0, RPT)], dbs_v[k])

    for k in range(3):
        for i in range(RPT // 16):
            dk = dbs_v[k][pl.ds(i * 16, 16)]
            adst_g = plsc.load_gather(adst_v, [dk])
            alpha = asrc_v[pl.ds(i * 16, 16)] + adst_g
            ex = jnp.exp(jnp.where(alpha >= 0, alpha, 0.2 * alpha))
            rowid = row0 + i * 16 + lanes
            exs_v[k][pl.ds(i * 16, 16)] = jnp.where(dk == rowid, 0.0, ex)

    for e in range(3 * RPC):
        denrows_v[e, :] = zeros16

    for f in range(NF):
        g = c * NF + f
        pltpu.sync_copy(accinit_hbm.at[g, pl.ds(row0, RPT), :],
                        acc_s.at[pl.ds(row0, RPT), :])
        if f == 0:
            pltpu.sync_copy(deninit_hbm.at[c, pl.ds(row0, RPT), :],
                            den_s.at[pl.ds(row0, RPT), :])
        pltpu.sync_copy(xw_hbm.at[g, pl.ds(row0, RPT), :], xw_v)
        plsc.subcore_barrier()

        def chunk(j, _):
            jr = j * RPC
            for k in range(3):
                for i in range(RPC // 16):
                    idxc_v[pl.ds(k * RPC + i * 16, 16)] = \
                        dbs_v[k][pl.ds(jr + i * 16, 16)]
            for r in range(RPC):
                xwb = [xw_v[jr + r, pl.ds(bb * 16, 16)]
                       for bb in range(FBLK // 16)]
                for k in range(3):
                    e = k * RPC + r
                    exb = plsc.load_gather(
                        exs_v[k], [jnp.full((16,), jr + r, jnp.int32)])
                    for bb in range(FBLK // 16):
                        msgs_v[e, pl.ds(bb * 16, 16)] = xwb[bb] * exb
            if f == 0:
                for k in range(3):
                    for i in range(RPC // 16):
                        e0 = k * RPC + i * 16
                        evec = e0 + lanes
                        exv = exs_v[k][pl.ds(jr + i * 16, 16)]
                        plsc.store_scatter(denrows_v, [evec, izeros16],
                                           exv)
                pltpu.sync_copy(denrows_v, den_s.at[idxc_v], add=True)
            pltpu.sync_copy(msgs_v, acc_s.at[idxc_v], add=True)
            return 0

        lax.fori_loop(0, NCH, chunk, 0)
        plsc.subcore_barrier()
        pltpu.sync_copy(acc_s.at[pl.ds(row0, RPT), :],
                        acc_out.at[c, f, pl.ds(row0, RPT), :])
        if f == 0:
            pltpu.sync_copy(den_s.at[pl.ds(row0, RPT), :],
                            den_out.at[c, pl.ds(row0, RPT), :])
        plsc.subcore_barrier()


@functools.lru_cache(maxsize=1)
def _sc_kernel():
    return functools.partial(
        pl.kernel,
        out_type=[
        jax.ShapeDtypeStruct((H, NF, N, FBLK), jnp.float32),
        jax.ShapeDtypeStruct((H, N, 16), jnp.float32),
    ],
        mesh=plsc.VectorSubcoreMesh(core_axis_name="c", subcore_axis_name="s"),
        compiler_params=pltpu.CompilerParams(needs_layout_passes=False),
        scratch_types=[
        pltpu.VMEM_SHARED((N, FBLK), jnp.float32),
            pltpu.VMEM_SHARED((N, 16), jnp.float32),
            pltpu.VMEM((RPT, FBLK), jnp.float32),
            pltpu.VMEM((3 * RPC, FBLK), jnp.float32),
            pltpu.VMEM((3 * RPC, 16), jnp.float32),
            pltpu.VMEM((RPT,), jnp.float32),
            pltpu.VMEM((RPT,), jnp.float32),
            pltpu.VMEM((RPT,), jnp.float32),
            pltpu.VMEM((RPT,), jnp.int32),
            pltpu.VMEM((RPT,), jnp.int32),
            pltpu.VMEM((RPT,), jnp.int32),
            pltpu.VMEM((RPT,), jnp.float32),
            pltpu.VMEM((N,), jnp.float32),
            pltpu.VMEM((3 * RPC,), jnp.int32),
        ],
    )(_sc_body)


def _sc_scatter(*args):
    return _sc_kernel()(*args)


def _post_body(acc_ref, d0_ref, d1_ref, p_ref, bias_ref, gamma_ref, beta_ref,
               out_ref):
    acc = acc_ref[...]
    d0 = d0_ref[...]
    d1 = d1_ref[...]
    a0 = jnp.concatenate([acc[0, f] for f in range(NF)], axis=1)
    a1 = jnp.concatenate([acc[1, f] for f in range(NF)], axis=1)
    out = 0.5 * (a0 / d0 + a1 / d1) + bias_ref[...]
    mu = jnp.mean(out, axis=1, keepdims=True)
    var = jnp.mean((out - mu) ** 2, axis=1, keepdims=True)
    out = (out - mu) / jnp.sqrt(var + 1e-5) * gamma_ref[...] + beta_ref[...]
    out_ref[...] = p_ref[...] + jnp.maximum(out, 0.0)


def _post(acc, den, prototypes, bias, gamma, beta):
    return pl.pallas_call(
        _post_body,
        grid=(N // QB,),
        in_specs=[
            pl.BlockSpec((H, NF, QB, FBLK), lambda i: (0, 0, i, 0)),
            pl.BlockSpec((QB, 1), lambda i: (i, 0)),
            pl.BlockSpec((QB, 1), lambda i: (i, 0)),
            pl.BlockSpec((QB, C), lambda i: (i, 0)),
            pl.BlockSpec((1, C), lambda i: (0, 0)),
            pl.BlockSpec((1, C), lambda i: (0, 0)),
            pl.BlockSpec((1, C), lambda i: (0, 0)),
        ],
        out_specs=pl.BlockSpec((QB, C), lambda i: (i, 0)),
        out_shape=jax.ShapeDtypeStruct((N, C), jnp.float32),
    )(acc, den[0, :, :1], den[1, :, :1], prototypes,
      bias.reshape(1, C), gamma.reshape(1, C), beta.reshape(1, C))


def kernel(prototypes, labels, W, att_src, att_dst, bias, gamma, beta):
    idx8, xw, accinit, deninit, asrcv, adstv = _fused(
        prototypes, W, att_src, att_dst)
    acc, den = _sc_scatter(xw, accinit, deninit, asrcv, adstv,
                           idx8[0], idx8[1], idx8[2])
    return _post(acc, den, prototypes, bias, gamma, beta)


# speedup vs baseline: 1.5851x; 1.0330x over previous
import functools

import jax
import jax.numpy as jnp
from jax import lax
from jax.experimental import pallas as pl
from jax.experimental.pallas import tpu as pltpu
from jax.experimental.pallas import tpu_sc as plsc

N = 8192
D = 256
H = 2
C = 256

QB = 512
KB = 2048

RPT = 512
RPC = 32
NCH = RPT // RPC
FBLK = 32
DSH = 5
DROWS = N // FBLK
NF = C // FBLK


def _top3_body(q_ref, k_ref, idx_ref):
    q = q_ref[...]
    qsq = jnp.sum(q * q, axis=1)

    def step(t, carry):
        v1, v2, v3, i1, i2, i3 = carry
        k = k_ref[pl.ds(t * KB, KB), :]
        ksq = jnp.sum(k * k, axis=1)
        dot = jax.lax.dot_general(
            q, k, (((1,), (1,)), ((), ())),
            preferred_element_type=jnp.float32)
        d2 = qsq[:, None] + ksq[None, :] - 2.0 * dot
        dist = jnp.sqrt(jnp.maximum(d2, 0.0))
        col = jax.lax.broadcasted_iota(jnp.int32, (QB, KB), 1) + t * KB

        def block_min(dmat):
            m = jnp.min(dmat, axis=1)
            idx = jnp.min(jnp.where(dmat == m[:, None], col, N), axis=1)
            dmat2 = jnp.where(col == idx[:, None], jnp.inf, dmat)
            return m, idx, dmat2

        m1, j1, dist = block_min(dist)
        m2, j2, dist = block_min(dist)
        m3, j3, dist = block_min(dist)

        def insert(m, j, v1, v2, v3, i1, i2, i3):
            c1 = m < v1
            c2 = m < v2
            c3 = m < v3
            nv3 = jnp.where(c3, jnp.where(c2, v2, m), v3)
            ni3 = jnp.where(c3, jnp.where(c2, i2, j), i3)
            nv2 = jnp.where(c2, jnp.where(c1, v1, m), v2)
            ni2 = jnp.where(c2, jnp.where(c1, i1, j), i2)
            nv1 = jnp.where(c1, m, v1)
            ni1 = jnp.where(c1, j, i1)
            return nv1, nv2, nv3, ni1, ni2, ni3

        v1, v2, v3, i1, i2, i3 = insert(m1, j1, v1, v2, v3, i1, i2, i3)
        v1, v2, v3, i1, i2, i3 = insert(m2, j2, v1, v2, v3, i1, i2, i3)
        v1, v2, v3, i1, i2, i3 = insert(m3, j3, v1, v2, v3, i1, i2, i3)
        return v1, v2, v3, i1, i2, i3

    inf = jnp.full((QB,), jnp.inf, dtype=jnp.float32)
    zero = jnp.zeros((QB,), dtype=jnp.int32)
    carry = (inf, inf, inf, zero, zero, zero)
    for t in range(N // KB):
        carry = step(t, carry)
    v1, v2, v3, i1, i2, i3 = carry
    idx_ref[...] = jnp.stack([i1, i2, i3, i1, i1, i1, i1, i1], axis=0)


def _leaky(x):
    return jnp.where(x >= 0, x, 0.2 * x)


def _fused_body(p_ref, k_ref, w_ref, asrcw_ref, adstw_ref,
                idx_ref, xw_ref, accinit_ref, deninit_ref,
                asrcv_ref, adstv_ref):
    _top3_body(p_ref, k_ref, idx_ref)
    p = p_ref[...]
    w = w_ref[...]
    xw = jnp.dot(p, w, preferred_element_type=jnp.float32)
    xw_ref[...] = jnp.stack(
        [xw[:, g * FBLK:(g + 1) * FBLK] for g in range(H * NF)], axis=0)
    asrcw = asrcw_ref[...]
    adstw = adstw_ref[...]
    h0 = xw[:, :C]
    h1 = xw[:, C:]
    as0 = jnp.sum(h0 * asrcw[0][None, :], axis=1)
    as1 = jnp.sum(h1 * asrcw[1][None, :], axis=1)
    ad0 = jnp.sum(h0 * adstw[0][None, :], axis=1)
    ad1 = jnp.sum(h1 * adstw[1][None, :], axis=1)
    es0 = jnp.exp(_leaky(as0 + ad0))
    es1 = jnp.exp(_leaky(as1 + ad1))
    accinit = jnp.concatenate([h0 * es0[:, None], h1 * es1[:, None]], axis=1)
    accinit_ref[...] = jnp.stack(
        [accinit[:, g * FBLK:(g + 1) * FBLK] for g in range(H * NF)], axis=0)
    col0 = jnp.stack([es0, es1], axis=0)[:, :, None]
    deninit_ref[...] = jnp.concatenate(
        [col0, jnp.zeros((H, QB, 15), jnp.float32)], axis=2)
    asrcv_ref[...] = jnp.stack([as0, as1], axis=0)[:, None, :]
    adstv_ref[...] = jnp.stack([ad0, ad1], axis=0)[:, None, :]


def _fused(prototypes, W, att_src, att_dst):
    return pl.pallas_call(
        _fused_body,
        grid=(N // QB,),
        in_specs=[
            pl.BlockSpec((QB, D), lambda i: (i, 0)),
            pl.BlockSpec((N, D), lambda i: (0, 0)),
            pl.BlockSpec((D, H * C), lambda i: (0, 0)),
            pl.BlockSpec((H, C), lambda i: (0, 0)),
            pl.BlockSpec((H, C), lambda i: (0, 0)),
        ],
        out_specs=[
            pl.BlockSpec((8, QB), lambda i: (0, i)),
            pl.BlockSpec((H * NF, QB, FBLK), lambda i: (0, i, 0)),
            pl.BlockSpec((H * NF, QB, FBLK), lambda i: (0, i, 0)),
            pl.BlockSpec((H, QB, 16), lambda i: (0, i, 0)),
            pl.BlockSpec((H, 1, QB), lambda i: (0, 0, i)),
            pl.BlockSpec((H, 1, QB), lambda i: (0, 0, i)),
        ],
        out_shape=[
            jax.ShapeDtypeStruct((8, N), jnp.int32),
            jax.ShapeDtypeStruct((H * NF, N, FBLK), jnp.float32),
            jax.ShapeDtypeStruct((H * NF, N, FBLK), jnp.float32),
            jax.ShapeDtypeStruct((H, N, 16), jnp.float32),
            jax.ShapeDtypeStruct((H, 1, N), jnp.float32),
            jax.ShapeDtypeStruct((H, 1, N), jnp.float32),
        ],
    )(prototypes, prototypes, W, att_src, att_dst)


def _sc_body(xw_hbm, accinit_hbm, deninit_hbm, asrc_hbm, adst_hbm,
             d0_hbm, d1_hbm, d2_hbm,
             acc_out, den_out,
             acc_s, den_s,
             xw_v, msgs_v, denrows_v, ex0_v, ex1_v, ex2_v,
             db0_v, db1_v, db2_v, asrc_v, adst_v, idxc_v):
    exs_v = (ex0_v, ex1_v, ex2_v)
    dbs_v = (db0_v, db1_v, db2_v)
    c = lax.axis_index("c")
    s = lax.axis_index("s")
    row0 = s * RPT
    lanes = lax.iota(jnp.int32, 16)
    zeros16 = jnp.zeros((16,), jnp.float32)
    izeros16 = jnp.zeros((16,), jnp.int32)

    pltpu.sync_copy(adst_hbm.at[c, 0], adst_v)
    pltpu.sync_copy(asrc_hbm.at[c, 0, pl.ds(row0, RPT)], asrc_v)
    for k, dbuf in enumerate((d0_hbm, d1_hbm, d2_hbm)):
        pltpu.sync_copy(dbuf.at[pl.ds(row0, RPT)], dbs_v[k])

    for k in range(3):
        for i in range(RPT // 16):
            dk = dbs_v[k][pl.ds(i * 16, 16)]
            adst_g = plsc.load_gather(adst_v, [dk])
            alpha = asrc_v[pl.ds(i * 16, 16)] + adst_g
            ex = jnp.exp(jnp.where(alpha >= 0, alpha, 0.2 * alpha))
            rowid = row0 + i * 16 + lanes
            exs_v[k][pl.ds(i * 16, 16)] = jnp.where(dk == rowid, 0.0, ex)

    for e in range(3 * RPC):
        denrows_v[e, :] = zeros16

    for f in range(NF):
        g = c * NF + f
        pltpu.sync_copy(accinit_hbm.at[g, pl.ds(row0, RPT), :],
                        acc_s.at[pl.ds(row0, RPT), :])
        if f == 0:
            pltpu.sync_copy(deninit_hbm.at[c, pl.ds(row0, RPT), :],
                            den_s.at[pl.ds(row0, RPT), :])
        pltpu.sync_copy(xw_hbm.at[g, pl.ds(row0, RPT), :], xw_v)
        plsc.subcore_barrier()

        def chunk(j, _):
            jr = j * RPC
            for k in range(3):
                for i in range(RPC // 16):
                    idxc_v[pl.ds(k * RPC + i * 16, 16)] = \
                        dbs_v[k][pl.ds(jr + i * 16, 16)]
            for r in range(RPC):
                xwb = [xw_v[jr + r, pl.ds(bb * 16, 16)]
                       for bb in range(FBLK // 16)]
                for k in range(3):
                    e = k * RPC + r
                    exb = plsc.load_gather(
                        exs_v[k], [jnp.full((16,), jr + r, jnp.int32)])
                    for bb in range(FBLK // 16):
                        msgs_v[e, pl.ds(bb * 16, 16)] = xwb[bb] * exb
            if f == 0:
                for k in range(3):
                    for i in range(RPC // 16):
                        e0 = k * RPC + i * 16
                        evec = e0 + lanes
                        exv = exs_v[k][pl.ds(jr + i * 16, 16)]
                        plsc.store_scatter(denrows_v, [evec, izeros16],
                                           exv)
                pltpu.sync_copy(denrows_v, den_s.at[idxc_v], add=True)
            pltpu.sync_copy(msgs_v, acc_s.at[idxc_v], add=True)
            return 0

        lax.fori_loop(0, NCH, chunk, 0)
        plsc.subcore_barrier()
        pltpu.sync_copy(acc_s.at[pl.ds(row0, RPT), :],
                        acc_out.at[c, f, pl.ds(row0, RPT), :])
        if f == 0:
            pltpu.sync_copy(den_s.at[pl.ds(row0, RPT), :],
                            den_out.at[c, pl.ds(row0, RPT), :])
        plsc.subcore_barrier()


@functools.lru_cache(maxsize=1)
def _sc_kernel():
    return functools.partial(
        pl.kernel,
        out_type=[
        jax.ShapeDtypeStruct((H, NF, N, FBLK), jnp.float32),
        jax.ShapeDtypeStruct((H, N, 16), jnp.float32),
    ],
        mesh=plsc.VectorSubcoreMesh(core_axis_name="c", subcore_axis_name="s"),
        compiler_params=pltpu.CompilerParams(needs_layout_passes=False),
        scratch_types=[
        pltpu.VMEM_SHARED((N, FBLK), jnp.float32),
            pltpu.VMEM_SHARED((N, 16), jnp.float32),
            pltpu.VMEM((RPT, FBLK), jnp.float32),
            pltpu.VMEM((3 * RPC, FBLK), jnp.float32),
            pltpu.VMEM((3 * RPC, 16), jnp.float32),
            pltpu.VMEM((RPT,), jnp.float32),
            pltpu.VMEM((RPT,), jnp.float32),
            pltpu.VMEM((RPT,), jnp.float32),
            pltpu.VMEM((RPT,), jnp.int32),
            pltpu.VMEM((RPT,), jnp.int32),
            pltpu.VMEM((RPT,), jnp.int32),
            pltpu.VMEM((RPT,), jnp.float32),
            pltpu.VMEM((N,), jnp.float32),
            pltpu.VMEM((3 * RPC,), jnp.int32),
        ],
    )(_sc_body)


def _sc_scatter(*args):
    return _sc_kernel()(*args)


def _post_body(acc_ref, d0_ref, d1_ref, p_ref, bias_ref, gamma_ref, beta_ref,
               out_ref):
    acc = acc_ref[...]
    d0 = d0_ref[...]
    d1 = d1_ref[...]
    a0 = jnp.concatenate([acc[0, f] for f in range(NF)], axis=1)
    a1 = jnp.concatenate([acc[1, f] for f in range(NF)], axis=1)
    out = 0.5 * (a0 / d0 + a1 / d1) + bias_ref[...]
    mu = jnp.mean(out, axis=1, keepdims=True)
    var = jnp.mean((out - mu) ** 2, axis=1, keepdims=True)
    out = (out - mu) / jnp.sqrt(var + 1e-5) * gamma_ref[...] + beta_ref[...]
    out_ref[...] = p_ref[...] + jnp.maximum(out, 0.0)


def _post(acc, den, prototypes, bias, gamma, beta):
    return pl.pallas_call(
        _post_body,
        grid=(N // QB,),
        in_specs=[
            pl.BlockSpec((H, NF, QB, FBLK), lambda i: (0, 0, i, 0)),
            pl.BlockSpec((QB, 1), lambda i: (i, 0)),
            pl.BlockSpec((QB, 1), lambda i: (i, 0)),
            pl.BlockSpec((QB, C), lambda i: (i, 0)),
            pl.BlockSpec((1, C), lambda i: (0, 0)),
            pl.BlockSpec((1, C), lambda i: (0, 0)),
            pl.BlockSpec((1, C), lambda i: (0, 0)),
        ],
        out_specs=pl.BlockSpec((QB, C), lambda i: (i, 0)),
        out_shape=jax.ShapeDtypeStruct((N, C), jnp.float32),
    )(acc, den[0, :, :1], den[1, :, :1], prototypes,
      bias.reshape(1, C), gamma.reshape(1, C), beta.reshape(1, C))


def kernel(prototypes, labels, W, att_src, att_dst, bias, gamma, beta):
    idx8, xw, accinit, deninit, asrcv, adstv = _fused(
        prototypes, W, att_src, att_dst)
    acc, den = _sc_scatter(xw, accinit, deninit, asrcv, adstv,
                           idx8[0], idx8[1], idx8[2])
    return _post(acc, den, prototypes, bias, gamma, beta)
